# baseline (speedup 1.0000x reference)
import functools

import jax
import jax.numpy as jnp
from jax import lax
from jax.experimental import pallas as pl
from jax.experimental.pallas import tpu as pltpu
from jax.experimental.pallas import tpu_sc as plsc

B, H, W, C = 32, 16, 16, 384
D, K = 256, 1024
N = B * C


def _argmin_body(x_ref, emb_ref, idx_ref, e2_ref, emb2_ref):
    b = pl.program_id(0)

    @pl.when(b == 0)
    def _():
        e2_ref[...] = jnp.sum(emb_ref[...] ** 2, axis=0, keepdims=True)
        emb2_ref[...] = emb_ref[...] + emb_ref[...]

    xb = x_ref[0].reshape(H * W, C)
    mm2 = lax.dot_general(
        xb, emb2_ref[...],
        dimension_numbers=(((0,), (0,)), ((), ())),
        preferred_element_type=jnp.float32,
    )
    f2 = jnp.sum(xb ** 2, axis=0)
    dists = (f2[:, None] + e2_ref[...]) - mm2
    m = jnp.min(dists, axis=1, keepdims=True)
    ks = lax.broadcasted_iota(jnp.int32, dists.shape, 1)
    idx_ref[0, 0, :] = jnp.min(jnp.where(dists == m, ks, K), axis=1)


def _tc_argmin(x, embeddings):
    return pl.pallas_call(
        _argmin_body,
        grid=(B,),
        in_specs=[
            pl.BlockSpec((1, H, W, C), lambda b: (b, 0, 0, 0)),
            pl.BlockSpec((D, K), lambda b: (0, 0)),
        ],
        out_specs=pl.BlockSpec((1, 1, C), lambda b: (b, 0, 0)),
        out_shape=jax.ShapeDtypeStruct((B, 1, C), jnp.int32),
        scratch_shapes=[pltpu.VMEM((1, K), jnp.float32),
                        pltpu.VMEM((D, K), jnp.float32)],
    )(x, embeddings)


@functools.lru_cache(maxsize=None)
def _make_sc_gather():
    info = plsc.get_sparse_core_info()
    nw = info.num_cores * info.num_subcores
    n_per_w = N // nw
    mesh = plsc.VectorSubcoreMesh(core_axis_name="c", subcore_axis_name="s")

    @functools.partial(
        pl.kernel, mesh=mesh,
        out_type=jax.ShapeDtypeStruct((N, D), jnp.float32),
        scratch_types=[
            pltpu.VMEM((n_per_w,), jnp.int32),
            pltpu.VMEM((n_per_w, D), jnp.float32),
            pltpu.SemaphoreType.DMA,
        ],
    )
    def gather(table_hbm, idx_hbm, out_hbm, idx_v, rows_v, sem):
        wid = lax.axis_index("s") * info.num_cores + lax.axis_index("c")
        base = wid * n_per_w
        pltpu.sync_copy(idx_hbm.at[pl.ds(base, n_per_w)], idx_v)
        pltpu.async_copy(table_hbm.at[idx_v], rows_v, sem).wait()
        pltpu.sync_copy(rows_v, out_hbm.at[pl.ds(base, n_per_w)])

    return gather


def kernel(x, embeddings):
    table = embeddings.T
    idx3 = _tc_argmin(x, embeddings)
    idx_flat = idx3.reshape(N)
    q_flat = _make_sc_gather()(table, idx_flat)
    quantised = q_flat.reshape(B, C, H, W).transpose(0, 2, 3, 1)
    discretised = idx3.reshape(B, C)
    return (quantised, discretised)


# speedup vs baseline: 1.0192x; 1.0192x over previous
import functools

import jax
import jax.numpy as jnp
from jax import lax
from jax.experimental import pallas as pl
from jax.experimental.pallas import tpu as pltpu
from jax.experimental.pallas import tpu_sc as plsc

B, H, W, C = 32, 16, 16, 384
D, K = 256, 1024
N = B * C


def _argmin_body(x_ref, emb_ref, idx_ref, embt_ref, e2_ref, emb2_ref):
    b = pl.program_id(0)

    @pl.when(b == 0)
    def _():
        e2_ref[...] = jnp.sum(emb_ref[...] ** 2, axis=0, keepdims=True)
        emb2_ref[...] = emb_ref[...] + emb_ref[...]
        embt_ref[...] = emb_ref[...].T

    xb = x_ref[0].reshape(H * W, C)
    mm2 = lax.dot_general(
        xb, emb2_ref[...],
        dimension_numbers=(((0,), (0,)), ((), ())),
        preferred_element_type=jnp.float32,
    )
    f2 = jnp.sum(xb ** 2, axis=0)
    dists = (f2[:, None] + e2_ref[...]) - mm2
    m = jnp.min(dists, axis=1, keepdims=True)
    ks = lax.broadcasted_iota(jnp.int32, dists.shape, 1)
    idx_ref[0, 0, :] = jnp.min(jnp.where(dists == m, ks, K), axis=1)


def _tc_argmin(x, embeddings):
    return pl.pallas_call(
        _argmin_body,
        grid=(B,),
        in_specs=[
            pl.BlockSpec((1, H, W, C), lambda b: (b, 0, 0, 0)),
            pl.BlockSpec((D, K), lambda b: (0, 0)),
        ],
        out_specs=[
            pl.BlockSpec((1, 1, C), lambda b: (b, 0, 0)),
            pl.BlockSpec((K, D), lambda b: (0, 0)),
        ],
        out_shape=[
            jax.ShapeDtypeStruct((B, 1, C), jnp.int32),
            jax.ShapeDtypeStruct((K, D), jnp.float32),
        ],
        scratch_shapes=[pltpu.VMEM((1, K), jnp.float32),
                        pltpu.VMEM((D, K), jnp.float32)],
    )(x, embeddings)


@functools.lru_cache(maxsize=None)
def _make_sc_gather():
    info = plsc.get_sparse_core_info()
    nw = info.num_cores * info.num_subcores
    n_per_w = N // nw
    mesh = plsc.VectorSubcoreMesh(core_axis_name="c", subcore_axis_name="s")

    @functools.partial(
        pl.kernel, mesh=mesh,
        out_type=jax.ShapeDtypeStruct((N, D), jnp.float32),
        scratch_types=[
            pltpu.VMEM((n_per_w,), jnp.int32),
            pltpu.VMEM((n_per_w, D), jnp.float32),
            pltpu.SemaphoreType.DMA,
        ],
    )
    def gather(table_hbm, idx_hbm, out_hbm, idx_v, rows_v, sem):
        wid = lax.axis_index("s") * info.num_cores + lax.axis_index("c")
        base = wid * n_per_w
        pltpu.sync_copy(idx_hbm.at[pl.ds(base, n_per_w)], idx_v)
        pltpu.async_copy(table_hbm.at[idx_v], rows_v, sem).wait()
        pltpu.sync_copy(rows_v, out_hbm.at[pl.ds(base, n_per_w)])

    return gather


def kernel(x, embeddings):
    idx3, table = _tc_argmin(x, embeddings)
    idx_flat = idx3.reshape(N)
    q_flat = _make_sc_gather()(table, idx_flat)
    quantised = q_flat.reshape(B, C, H, W).transpose(0, 2, 3, 1)
    discretised = idx3.reshape(B, C)
    return (quantised, discretised)


# speedup vs baseline: 1.1843x; 1.1620x over previous
import functools

import jax
import jax.numpy as jnp
from jax import lax
from jax.experimental import pallas as pl
from jax.experimental.pallas import tpu as pltpu
from jax.experimental.pallas import tpu_sc as plsc

B, H, W, C = 32, 16, 16, 384
D, K = 256, 1024
N = B * C


def _argmin_body(x_ref, emb_ref, idx_ref, embt_ref, e2_ref, emb2_ref):
    b = pl.program_id(0)

    @pl.when(b == 0)
    def _():
        e2_ref[...] = jnp.sum(emb_ref[...] ** 2, axis=0, keepdims=True)
        emb2_ref[...] = emb_ref[...] + emb_ref[...]
        embt_ref[...] = emb_ref[...].T.reshape(2 * K, 128)

    xb = x_ref[0].reshape(H * W, C)
    mm2 = lax.dot_general(
        xb, emb2_ref[...],
        dimension_numbers=(((0,), (0,)), ((), ())),
        preferred_element_type=jnp.float32,
    )
    f2 = jnp.sum(xb ** 2, axis=0)
    dists = (f2[:, None] + e2_ref[...]) - mm2
    m = jnp.min(dists, axis=1, keepdims=True)
    ks = lax.broadcasted_iota(jnp.int32, dists.shape, 1)
    idx_ref[0, 0, :] = jnp.min(jnp.where(dists == m, ks, K), axis=1)


def _tc_argmin(x, embeddings):
    return pl.pallas_call(
        _argmin_body,
        grid=(B,),
        in_specs=[
            pl.BlockSpec((1, H, W, C), lambda b: (b, 0, 0, 0)),
            pl.BlockSpec((D, K), lambda b: (0, 0)),
        ],
        out_specs=[
            pl.BlockSpec((1, 1, C), lambda b: (b, 0, 0)),
            pl.BlockSpec((2 * K, 128), lambda b: (0, 0)),
        ],
        out_shape=[
            jax.ShapeDtypeStruct((B, 1, C), jnp.int32),
            jax.ShapeDtypeStruct((2 * K, 128), jnp.float32),
        ],
        scratch_shapes=[pltpu.VMEM((1, K), jnp.float32),
                        pltpu.VMEM((D, K), jnp.float32)],
    )(x, embeddings)


@functools.lru_cache(maxsize=None)
def _make_sc_gather():
    info = plsc.get_sparse_core_info()
    nw = info.num_cores * info.num_subcores
    n_per_w = N // nw
    mesh = plsc.VectorSubcoreMesh(core_axis_name="c", subcore_axis_name="s")

    @functools.partial(
        pl.kernel, mesh=mesh,
        out_type=jax.ShapeDtypeStruct((2 * N, 128), jnp.float32),
        scratch_types=[
            pltpu.VMEM((n_per_w + 16,), jnp.int32),
            pltpu.VMEM((2 * n_per_w,), jnp.int32),
            pltpu.VMEM((2 * n_per_w, 128), jnp.float32),
            pltpu.SemaphoreType.DMA,
        ],
    )
    def gather(table_hbm, idx_hbm, out_hbm, idx_v, idx2_v, rows_v, sem):
        wid = lax.axis_index("s") * info.num_cores + lax.axis_index("c")
        base = wid * n_per_w
        pltpu.sync_copy(idx_hbm.at[pl.ds(base, n_per_w)],
                        idx_v.at[pl.ds(0, n_per_w)])

        lanes = lax.iota(jnp.int32, 16)
        dup = lanes >> 1
        parity = lanes & 1

        def build(g, carry):
            k = idx_v[pl.ds(g * 8, 16)]
            e = lax.gather(
                k, dup[:, None],
                lax.GatherDimensionNumbers(offset_dims=(),
                                           collapsed_slice_dims=(0,),
                                           start_index_map=(0,)),
                (1,), mode=lax.GatherScatterMode.PROMISE_IN_BOUNDS)
            idx2_v[pl.ds(g * 16, 16)] = e + e + parity
            return carry

        lax.fori_loop(0, (2 * n_per_w) // 16, build, 0)
        pltpu.async_copy(table_hbm.at[idx2_v], rows_v, sem).wait()
        pltpu.sync_copy(rows_v, out_hbm.at[pl.ds(2 * base, 2 * n_per_w)])

    return gather


def _finalize_body(q_ref, out_ref):
    a = q_ref[...].reshape(C, D)
    out_ref[0] = a.T.reshape(H, W, C)


def _tc_finalize(q2):
    return pl.pallas_call(
        _finalize_body,
        grid=(B,),
        in_specs=[pl.BlockSpec((2 * C, 128), lambda b: (b, 0))],
        out_specs=pl.BlockSpec((1, H, W, C), lambda b: (b, 0, 0, 0)),
        out_shape=jax.ShapeDtypeStruct((B, H, W, C), jnp.float32),
    )(q2)


def kernel(x, embeddings):
    idx3, table2 = _tc_argmin(x, embeddings)
    idx_flat = idx3.reshape(N)
    q2 = _make_sc_gather()(table2, idx_flat)
    quantised = _tc_finalize(q2)
    discretised = idx3.reshape(B, C)
    return (quantised, discretised)


# speedup vs baseline: 1.2397x; 1.0469x over previous
import functools

import jax
import jax.numpy as jnp
from jax import lax
from jax.experimental import pallas as pl
from jax.experimental.pallas import tpu as pltpu
from jax.experimental.pallas import tpu_sc as plsc

B, H, W, C = 32, 16, 16, 384
D, K = 256, 1024
N = B * C


def _argmin_body(x_ref, emb_ref, idx_ref, embt_ref, e2_ref, emb2_ref):
    b = pl.program_id(0)

    @pl.when(b == 0)
    def _():
        e2_ref[...] = jnp.sum(emb_ref[...] ** 2, axis=0, keepdims=True)
        emb2_ref[...] = emb_ref[...] + emb_ref[...]
        embt_ref[...] = emb_ref[...].T.reshape(2 * K, 128)

    xb = x_ref[0]
    mm2 = lax.dot_general(
        xb, emb2_ref[...],
        dimension_numbers=(((0,), (0,)), ((), ())),
        preferred_element_type=jnp.float32,
    )
    f2 = jnp.sum(xb ** 2, axis=0)
    dists = (f2[:, None] + e2_ref[...]) - mm2
    m = jnp.min(dists, axis=1, keepdims=True)
    ks = lax.broadcasted_iota(jnp.int32, dists.shape, 1).astype(jnp.float32)
    sel = jnp.where(dists == m, ks, jnp.float32(K))
    idx_ref[0, 0, :] = jnp.min(sel, axis=1).astype(jnp.int32)


def _tc_argmin(x, embeddings):
    return pl.pallas_call(
        _argmin_body,
        grid=(B,),
        in_specs=[
            pl.BlockSpec((1, D, C), lambda b: (b, 0, 0)),
            pl.BlockSpec((D, K), lambda b: (0, 0)),
        ],
        out_specs=[
            pl.BlockSpec((1, 1, C), lambda b: (b, 0, 0)),
            pl.BlockSpec((2 * K, 128), lambda b: (0, 0)),
        ],
        out_shape=[
            jax.ShapeDtypeStruct((B, 1, C), jnp.int32),
            jax.ShapeDtypeStruct((2 * K, 128), jnp.float32),
        ],
        scratch_shapes=[pltpu.VMEM((1, K), jnp.float32),
                        pltpu.VMEM((D, K), jnp.float32)],
    )(x, embeddings)


@functools.lru_cache(maxsize=None)
def _make_sc_gather():
    info = plsc.get_sparse_core_info()
    nw = info.num_cores * info.num_subcores
    n_per_w = N // nw
    mesh = plsc.VectorSubcoreMesh(core_axis_name="c", subcore_axis_name="s")

    @functools.partial(
        pl.kernel, mesh=mesh,
        out_type=jax.ShapeDtypeStruct((2 * N, 128), jnp.float32),
        scratch_types=[
            pltpu.VMEM((n_per_w + 16,), jnp.int32),
            pltpu.VMEM((2 * n_per_w,), jnp.int32),
            pltpu.VMEM((2 * n_per_w, 128), jnp.float32),
            pltpu.SemaphoreType.DMA,
        ],
    )
    def gather(table_hbm, idx_hbm, out_hbm, idx_v, idx2_v, rows_v, sem):
        wid = lax.axis_index("s") * info.num_cores + lax.axis_index("c")
        base = wid * n_per_w
        pltpu.sync_copy(idx_hbm.at[pl.ds(base, n_per_w)],
                        idx_v.at[pl.ds(0, n_per_w)])

        lanes = lax.iota(jnp.int32, 16)
        dup = lanes >> 1
        parity = lanes & 1

        def build(g, carry):
            k = idx_v[pl.ds(g * 8, 16)]
            e = lax.gather(
                k, dup[:, None],
                lax.GatherDimensionNumbers(offset_dims=(),
                                           collapsed_slice_dims=(0,),
                                           start_index_map=(0,)),
                (1,), mode=lax.GatherScatterMode.PROMISE_IN_BOUNDS)
            idx2_v[pl.ds(g * 16, 16)] = e + e + parity
            return carry

        lax.fori_loop(0, (2 * n_per_w) // 16, build, 0)
        pltpu.async_copy(table_hbm.at[idx2_v], rows_v, sem).wait()
        pltpu.sync_copy(rows_v, out_hbm.at[pl.ds(2 * base, 2 * n_per_w)])

    return gather


def _finalize_body(q_ref, out_ref):
    a = q_ref[...].reshape(C, D)
    out_ref[0] = a.T.reshape(H, W, C)


def _tc_finalize(q2):
    return pl.pallas_call(
        _finalize_body,
        grid=(B,),
        in_specs=[pl.BlockSpec((2 * C, 128), lambda b: (b, 0))],
        out_specs=pl.BlockSpec((1, H, W, C), lambda b: (b, 0, 0, 0)),
        out_shape=jax.ShapeDtypeStruct((B, H, W, C), jnp.float32),
    )(q2)


def kernel(x, embeddings):
    x_r = x.reshape(B, H * W, C)
    idx3, table2 = _tc_argmin(x_r, embeddings)
    idx_flat = idx3.reshape(N)
    q2 = _make_sc_gather()(table2, idx_flat)
    quantised = _tc_finalize(q2)
    discretised = idx3.reshape(B, C)
    return (quantised, discretised)
